# baseline (speedup 1.0000x reference)
import functools

import jax
import jax.numpy as jnp
from jax import lax
from jax.experimental import pallas as pl
from jax.experimental.pallas import tpu as pltpu
from jax.experimental.pallas import tpu_sc as plsc

R = 128
N = 32768
L = 16
NW = 32
SC_ROWS = 64
TC_ROWS = R - SC_ROWS
ROWS_PER_W = SC_ROWS // NW
ACC = 8
STEPS = N // (L * ACC)

TC_BLK = 16


def _bcast_max(x):
    fwd = plsc.cummax(x)
    bwd = lax.rev(plsc.cummax(lax.rev(x, (0,))), (0,))
    return jnp.maximum(fwd, bwd)


def _combine(a1, a2, b1, b2):
    n1 = jnp.maximum(a1, b1)
    n2 = jnp.maximum(jnp.minimum(a1, b1), jnp.maximum(a2, b2))
    return n1, n2


_mesh = plsc.VectorSubcoreMesh(core_axis_name="c", subcore_axis_name="s")


NBUF = 3
CHUNKS_PER_ROW = 4
CHUNK = N // CHUNKS_PER_ROW
CSTEPS = CHUNK // (L * ACC)
TOTAL_CHUNKS = ROWS_PER_W * CHUNKS_PER_ROW


SC_OUT = SC_ROWS + 8


@functools.partial(
    pl.kernel,
    mesh=_mesh,
    out_type=jax.ShapeDtypeStruct((SC_OUT,), jnp.float32),
    scratch_types=[
        pltpu.VMEM((CHUNK,), jnp.float32),
        pltpu.VMEM((CHUNK,), jnp.float32),
        pltpu.VMEM((CHUNK,), jnp.float32),
        pltpu.VMEM((L,), jnp.float32),
        pltpu.SemaphoreType.DMA,
        pltpu.SemaphoreType.DMA,
        pltpu.SemaphoreType.DMA,
    ],
    compiler_params=pltpu.CompilerParams(needs_layout_passes=False),
)
def _top2_margin_sc(logits_hbm, out_hbm, buf0, buf1, buf2, res_v,
                    sem0, sem1, sem2):
    cid = lax.axis_index("c")
    sid = lax.axis_index("s")
    wid = cid * 16 + sid
    base = wid * ROWS_PER_W
    sems = (sem0, sem1, sem2)
    bufs = (buf0, buf1, buf2)

    def issue(g):
        row_ref = logits_hbm.at[base + g // CHUNKS_PER_ROW]
        src = row_ref.at[pl.ds((g % CHUNKS_PER_ROW) * CHUNK, CHUNK)]
        return pltpu.async_copy(src, bufs[g % NBUF], sems[g % NBUF])

    copies = {}
    for g in range(min(NBUF - 1, TOTAL_CHUNKS)):
        copies[g] = issue(g)

    res = jnp.zeros((L,), jnp.float32)
    neg = jnp.full((L,), -jnp.inf, jnp.float32)
    m1l = m2l = None
    for g in range(TOTAL_CHUNKS):
        if g + NBUF - 1 < TOTAL_CHUNKS:
            copies[g + NBUF - 1] = issue(g + NBUF - 1)
        copies[g].wait()
        chunk_ref = bufs[g % NBUF]

        if g % CHUNKS_PER_ROW == 0:
            m1l = tuple([neg] * ACC)
            m2l = tuple([neg] * ACC)

        def body(i, carry, chunk_ref=chunk_ref):
            m1s, m2s = carry
            n1, n2 = [], []
            for a in range(ACC):
                x = chunk_ref[pl.ds((i * ACC + a) * L, L)]
                n1.append(jnp.maximum(m1s[a], x))
                n2.append(jnp.maximum(m2s[a], jnp.minimum(m1s[a], x)))
            return tuple(n1), tuple(n2)

        m1l, m2l = lax.fori_loop(0, CSTEPS, body, (m1l, m2l))

        if g % CHUNKS_PER_ROW == CHUNKS_PER_ROW - 1:
            j = g // CHUNKS_PER_ROW
            p1, p2 = list(m1l), list(m2l)
            while len(p1) > 1:
                n1, n2 = [], []
                for a in range(0, len(p1), 2):
                    c1, c2 = _combine(p1[a], p2[a], p1[a + 1], p2[a + 1])
                    n1.append(c1)
                    n2.append(c2)
                p1, p2 = n1, n2
            m1, m2 = p1[0], p2[0]

            s1v = _bcast_max(m1)
            maskv = m1 == s1v
            cntv = plsc.all_reduce_population_count(maskv)
            t = jnp.where(maskv, m2, m1)
            s2v = jnp.where(cntv >= 2, s1v, _bcast_max(t))
            margin = s1v - s2v

            lane = lax.iota(jnp.int32, L)
            res = jnp.where(lane == j, margin, res)

    res_v[...] = res
    lane = lax.iota(jnp.int32, L)
    idx = jnp.where(lane < ROWS_PER_W, base + lane, SC_ROWS)
    pltpu.async_copy(res_v, out_hbm.at[idx], sems[0]).wait()


TC_CHUNK = 1024


def _top2_margin_tc_body(x_ref, o_ref):
    neg = jnp.full((TC_BLK, TC_CHUNK), -jnp.inf, jnp.float32)

    def step(c, carry):
        m1, m2 = carry
        x = x_ref[:, pl.ds(c * TC_CHUNK, TC_CHUNK)]
        n1 = jnp.maximum(m1, x)
        n2 = jnp.maximum(m2, jnp.minimum(m1, x))
        return n1, n2

    m1, m2 = lax.fori_loop(0, N // TC_CHUNK, step, (neg, neg))
    s1 = jnp.max(m1, axis=1, keepdims=True)
    eq = m1 == s1
    cnt = jnp.sum(eq.astype(jnp.float32), axis=1, keepdims=True)
    t = jnp.where(eq, m2, m1)
    s2 = jnp.max(t, axis=1, keepdims=True)
    margin = jnp.where(cnt >= 2.0, jnp.zeros_like(s1), s1 - s2)
    o_ref[...] = jnp.broadcast_to(margin, (TC_BLK, 128))


_tc_call = pl.pallas_call(
    _top2_margin_tc_body,
    grid=(TC_ROWS // TC_BLK,),
    in_specs=[
        pl.BlockSpec((TC_BLK, N), lambda i: (i + SC_ROWS // TC_BLK, 0)),
    ],
    out_specs=pl.BlockSpec((TC_BLK, 128), lambda i: (i, 0)),
    out_shape=jax.ShapeDtypeStruct((TC_ROWS, 128), jnp.float32),
)


def kernel(logits):
    sc = _top2_margin_sc(logits)
    tc = _tc_call(logits)
    return jnp.concatenate([sc[:SC_ROWS], tc[:, 0]])


# speedup vs baseline: 3.7488x; 3.7488x over previous
import functools

import jax
import jax.numpy as jnp
from jax import lax
from jax.experimental import pallas as pl
from jax.experimental.pallas import tpu as pltpu
from jax.experimental.pallas import tpu_sc as plsc

R = 128
N = 32768
L = 16
NW = 32
SC_ROWS = 64
TC_ROWS = R - SC_ROWS
ROWS_PER_W = SC_ROWS // NW
ACC = 8
STEPS = N // (L * ACC)

TC_BLK = 16


def _bcast_max(x):
    fwd = plsc.cummax(x)
    bwd = lax.rev(plsc.cummax(lax.rev(x, (0,))), (0,))
    return jnp.maximum(fwd, bwd)


def _combine(a1, a2, b1, b2):
    n1 = jnp.maximum(a1, b1)
    n2 = jnp.maximum(jnp.minimum(a1, b1), jnp.maximum(a2, b2))
    return n1, n2


_mesh = plsc.VectorSubcoreMesh(core_axis_name="c", subcore_axis_name="s")


NBUF = 4
CHUNKS_PER_ROW = 8
CHUNK = N // CHUNKS_PER_ROW
CSTEPS = CHUNK // (L * ACC)
TOTAL_CHUNKS = ROWS_PER_W * CHUNKS_PER_ROW


@functools.partial(
    pl.kernel,
    mesh=_mesh,
    out_type=jax.ShapeDtypeStruct((SC_ROWS, L), jnp.float32),
    scratch_types=[
        pltpu.VMEM((CHUNK,), jnp.float32),
        pltpu.VMEM((CHUNK,), jnp.float32),
        pltpu.VMEM((CHUNK,), jnp.float32),
        pltpu.VMEM((CHUNK,), jnp.float32),
        pltpu.VMEM((L,), jnp.float32),
        pltpu.SemaphoreType.DMA,
        pltpu.SemaphoreType.DMA,
        pltpu.SemaphoreType.DMA,
        pltpu.SemaphoreType.DMA,
    ],
    compiler_params=pltpu.CompilerParams(needs_layout_passes=False),
)
def _top2_margin_sc(logits_hbm, out_hbm, buf0, buf1, buf2, buf3, res_v,
                    sem0, sem1, sem2, sem3):
    cid = lax.axis_index("c")
    sid = lax.axis_index("s")
    wid = cid * 16 + sid
    base = wid * ROWS_PER_W
    sems = (sem0, sem1, sem2, sem3)
    bufs = (buf0, buf1, buf2, buf3)

    def issue(g):
        row_ref = logits_hbm.at[base + g // CHUNKS_PER_ROW]
        src = row_ref.at[pl.ds((g % CHUNKS_PER_ROW) * CHUNK, CHUNK)]
        return pltpu.async_copy(src, bufs[g % NBUF], sems[g % NBUF])

    copies = {}
    for g in range(min(NBUF - 1, TOTAL_CHUNKS)):
        copies[g] = issue(g)

    neg = jnp.full((L,), -jnp.inf, jnp.float32)
    m1l = m2l = None
    for g in range(TOTAL_CHUNKS):
        if g + NBUF - 1 < TOTAL_CHUNKS:
            copies[g + NBUF - 1] = issue(g + NBUF - 1)
        copies[g].wait()
        chunk_ref = bufs[g % NBUF]

        if g % CHUNKS_PER_ROW == 0:
            m1l = tuple([neg] * ACC)
            m2l = tuple([neg] * ACC)

        def body(i, carry, chunk_ref=chunk_ref):
            m1s, m2s = carry
            n1, n2 = [], []
            for a in range(ACC):
                x = chunk_ref[pl.ds((i * ACC + a) * L, L)]
                n1.append(jnp.maximum(m1s[a], x))
                n2.append(jnp.maximum(m2s[a], jnp.minimum(m1s[a], x)))
            return tuple(n1), tuple(n2)

        m1l, m2l = lax.fori_loop(0, CSTEPS, body, (m1l, m2l))

        if g % CHUNKS_PER_ROW == CHUNKS_PER_ROW - 1:
            j = g // CHUNKS_PER_ROW
            p1, p2 = list(m1l), list(m2l)
            while len(p1) > 1:
                n1, n2 = [], []
                for a in range(0, len(p1), 2):
                    c1, c2 = _combine(p1[a], p2[a], p1[a + 1], p2[a + 1])
                    n1.append(c1)
                    n2.append(c2)
                p1, p2 = n1, n2
            m1, m2 = p1[0], p2[0]

            s1v = _bcast_max(m1)
            maskv = m1 == s1v
            cntv = plsc.all_reduce_population_count(maskv)
            t = jnp.where(maskv, m2, m1)
            s2v = jnp.where(cntv >= 2, s1v, _bcast_max(t))
            margin = s1v - s2v

            res_v[...] = margin
            pltpu.sync_copy(res_v, out_hbm.at[base + j])


TC_CHUNK = 1024


def _top2_margin_tc_body(x_ref, o_ref):
    neg = jnp.full((TC_BLK, TC_CHUNK), -jnp.inf, jnp.float32)

    def step(c, carry):
        m1, m2 = carry
        x = x_ref[:, pl.ds(c * TC_CHUNK, TC_CHUNK)]
        n1 = jnp.maximum(m1, x)
        n2 = jnp.maximum(m2, jnp.minimum(m1, x))
        return n1, n2

    m1, m2 = lax.fori_loop(0, N // TC_CHUNK, step, (neg, neg))
    s1 = jnp.max(m1, axis=1, keepdims=True)
    eq = m1 == s1
    cnt = jnp.sum(eq.astype(jnp.float32), axis=1, keepdims=True)
    t = jnp.where(eq, m2, m1)
    s2 = jnp.max(t, axis=1, keepdims=True)
    margin = jnp.where(cnt >= 2.0, jnp.zeros_like(s1), s1 - s2)
    o_ref[...] = jnp.broadcast_to(margin, (TC_BLK, 128))


_tc_call = pl.pallas_call(
    _top2_margin_tc_body,
    grid=(TC_ROWS // TC_BLK,),
    in_specs=[
        pl.BlockSpec((TC_BLK, N), lambda i: (i + SC_ROWS // TC_BLK, 0)),
    ],
    out_specs=pl.BlockSpec((TC_BLK, 128), lambda i: (i, 0)),
    out_shape=jax.ShapeDtypeStruct((TC_ROWS, 128), jnp.float32),
)


def kernel(logits):
    sc = _top2_margin_sc(logits)
    tc = _tc_call(logits)
    return jnp.concatenate([sc[:, 0], tc[:, 0]])


# speedup vs baseline: 3.8189x; 1.0187x over previous
import functools

import jax
import jax.numpy as jnp
from jax import lax
from jax.experimental import pallas as pl
from jax.experimental.pallas import tpu as pltpu
from jax.experimental.pallas import tpu_sc as plsc

R = 128
N = 32768
L = 16
NW = 32
SC_ROWS = 64
TC_ROWS = R - SC_ROWS
ROWS_PER_W = SC_ROWS // NW
ACC = 8
STEPS = N // (L * ACC)

TC_BLK = 16


def _bcast_max(x):
    fwd = plsc.cummax(x)
    bwd = lax.rev(plsc.cummax(lax.rev(x, (0,))), (0,))
    return jnp.maximum(fwd, bwd)


def _combine(a1, a2, b1, b2):
    n1 = jnp.maximum(a1, b1)
    n2 = jnp.maximum(jnp.minimum(a1, b1), jnp.maximum(a2, b2))
    return n1, n2


_mesh = plsc.VectorSubcoreMesh(core_axis_name="c", subcore_axis_name="s")


NBUF = 3
CHUNKS_PER_ROW = 4
CHUNK = N // CHUNKS_PER_ROW
CSTEPS = CHUNK // (L * ACC)
TOTAL_CHUNKS = ROWS_PER_W * CHUNKS_PER_ROW


@functools.partial(
    pl.kernel,
    mesh=_mesh,
    out_type=jax.ShapeDtypeStruct((NW, L), jnp.float32),
    scratch_types=[
        pltpu.VMEM((CHUNK,), jnp.float32),
        pltpu.VMEM((CHUNK,), jnp.float32),
        pltpu.VMEM((CHUNK,), jnp.float32),
        pltpu.VMEM((L,), jnp.float32),
        pltpu.SemaphoreType.DMA,
        pltpu.SemaphoreType.DMA,
        pltpu.SemaphoreType.DMA,
    ],
    compiler_params=pltpu.CompilerParams(needs_layout_passes=False),
)
def _top2_margin_sc(logits_hbm, out_hbm, buf0, buf1, buf2, res_v,
                    sem0, sem1, sem2):
    cid = lax.axis_index("c")
    sid = lax.axis_index("s")
    wid = cid * 16 + sid
    base = wid * ROWS_PER_W
    sems = (sem0, sem1, sem2)
    bufs = (buf0, buf1, buf2)

    def issue(g):
        row_ref = logits_hbm.at[base + g // CHUNKS_PER_ROW]
        src = row_ref.at[pl.ds((g % CHUNKS_PER_ROW) * CHUNK, CHUNK)]
        return pltpu.async_copy(src, bufs[g % NBUF], sems[g % NBUF])

    copies = {}
    for g in range(min(NBUF - 1, TOTAL_CHUNKS)):
        copies[g] = issue(g)

    res = jnp.zeros((L,), jnp.float32)
    neg = jnp.full((L,), -jnp.inf, jnp.float32)
    m1l = m2l = None
    for g in range(TOTAL_CHUNKS):
        if g + NBUF - 1 < TOTAL_CHUNKS:
            copies[g + NBUF - 1] = issue(g + NBUF - 1)
        copies[g].wait()
        chunk_ref = bufs[g % NBUF]

        if g % CHUNKS_PER_ROW == 0:
            m1l = tuple([neg] * ACC)
            m2l = tuple([neg] * ACC)

        @plsc.parallel_loop(0, CSTEPS, unroll=2, carry=(m1l, m2l))
        def _chunk_loop(i, carry, chunk_ref=chunk_ref):
            m1s, m2s = carry
            n1, n2 = [], []
            for a in range(ACC):
                x = chunk_ref[pl.ds((i * ACC + a) * L, L)]
                n1.append(jnp.maximum(m1s[a], x))
                n2.append(jnp.maximum(m2s[a], jnp.minimum(m1s[a], x)))
            return tuple(n1), tuple(n2)

        m1l, m2l = _chunk_loop

        if g % CHUNKS_PER_ROW == CHUNKS_PER_ROW - 1:
            j = g // CHUNKS_PER_ROW
            p1, p2 = list(m1l), list(m2l)
            while len(p1) > 1:
                n1, n2 = [], []
                for a in range(0, len(p1), 2):
                    c1, c2 = _combine(p1[a], p2[a], p1[a + 1], p2[a + 1])
                    n1.append(c1)
                    n2.append(c2)
                p1, p2 = n1, n2
            m1, m2 = p1[0], p2[0]

            s1v = _bcast_max(m1)
            maskv = m1 == s1v
            cntv = plsc.all_reduce_population_count(maskv)
            t = jnp.where(maskv, m2, m1)
            s2v = jnp.where(cntv >= 2, s1v, _bcast_max(t))
            margin = s1v - s2v

            lane = lax.iota(jnp.int32, L)
            res = jnp.where(lane == j, margin, res)

    res_v[...] = res
    pltpu.sync_copy(res_v, out_hbm.at[wid])


TC_CHUNK = 1024


def _top2_margin_tc_body(x_ref, o_ref):
    neg = jnp.full((TC_BLK, TC_CHUNK), -jnp.inf, jnp.float32)

    def step(c, carry):
        m1, m2 = carry
        x = x_ref[:, pl.ds(c * TC_CHUNK, TC_CHUNK)]
        n1 = jnp.maximum(m1, x)
        n2 = jnp.maximum(m2, jnp.minimum(m1, x))
        return n1, n2

    m1, m2 = lax.fori_loop(0, N // TC_CHUNK, step, (neg, neg))
    s1 = jnp.max(m1, axis=1, keepdims=True)
    eq = m1 == s1
    cnt = jnp.sum(eq.astype(jnp.float32), axis=1, keepdims=True)
    t = jnp.where(eq, m2, m1)
    s2 = jnp.max(t, axis=1, keepdims=True)
    margin = jnp.where(cnt >= 2.0, jnp.zeros_like(s1), s1 - s2)
    o_ref[...] = jnp.broadcast_to(margin, (TC_BLK, 128))


_tc_call = pl.pallas_call(
    _top2_margin_tc_body,
    grid=(TC_ROWS // TC_BLK,),
    in_specs=[
        pl.BlockSpec((TC_BLK, N), lambda i: (i + SC_ROWS // TC_BLK, 0)),
    ],
    out_specs=pl.BlockSpec((TC_BLK, 128), lambda i: (i, 0)),
    out_shape=jax.ShapeDtypeStruct((TC_ROWS, 128), jnp.float32),
)


def kernel(logits):
    sc = _top2_margin_sc(logits)
    tc = _tc_call(logits)
    return jnp.concatenate([sc[:, :ROWS_PER_W].reshape(SC_ROWS), tc[:, 0]])


# speedup vs baseline: 3.8899x; 1.0186x over previous
import functools

import jax
import jax.numpy as jnp
from jax import lax
from jax.experimental import pallas as pl
from jax.experimental.pallas import tpu as pltpu
from jax.experimental.pallas import tpu_sc as plsc

R = 128
N = 32768
L = 16
NW = 32
SC_ROWS = 64
TC_ROWS = R - SC_ROWS
ROWS_PER_W = SC_ROWS // NW
ACC = 8
STEPS = N // (L * ACC)

TC_BLK = 16


def _bcast_max(x):
    fwd = plsc.cummax(x)
    bwd = lax.rev(plsc.cummax(lax.rev(x, (0,))), (0,))
    return jnp.maximum(fwd, bwd)


def _combine(a1, a2, b1, b2):
    n1 = jnp.maximum(a1, b1)
    n2 = jnp.maximum(jnp.minimum(a1, b1), jnp.maximum(a2, b2))
    return n1, n2


_mesh = plsc.VectorSubcoreMesh(core_axis_name="c", subcore_axis_name="s")


NBUF = 3
CHUNK = 8192
SC_COLS = 24576
CHUNKS_PER_ROW = SC_COLS // CHUNK
CSTEPS = CHUNK // (L * ACC)
TOTAL_CHUNKS = ROWS_PER_W * CHUNKS_PER_ROW


@functools.partial(
    pl.kernel,
    mesh=_mesh,
    out_type=jax.ShapeDtypeStruct((NW, L), jnp.float32),
    scratch_types=[
        pltpu.VMEM((CHUNK,), jnp.float32),
        pltpu.VMEM((CHUNK,), jnp.float32),
        pltpu.VMEM((CHUNK,), jnp.float32),
        pltpu.VMEM((L,), jnp.float32),
        pltpu.SemaphoreType.DMA,
        pltpu.SemaphoreType.DMA,
        pltpu.SemaphoreType.DMA,
    ],
    compiler_params=pltpu.CompilerParams(needs_layout_passes=False),
)
def _top2_margin_sc(logits_hbm, out_hbm, buf0, buf1, buf2, res_v,
                    sem0, sem1, sem2):
    cid = lax.axis_index("c")
    sid = lax.axis_index("s")
    wid = cid * 16 + sid
    base = wid * ROWS_PER_W
    sems = (sem0, sem1, sem2)
    bufs = (buf0, buf1, buf2)

    def issue(g):
        row_ref = logits_hbm.at[base + g // CHUNKS_PER_ROW]
        src = row_ref.at[pl.ds((g % CHUNKS_PER_ROW) * CHUNK, CHUNK)]
        return pltpu.async_copy(src, bufs[g % NBUF], sems[g % NBUF])

    copies = {}
    for g in range(min(NBUF - 1, TOTAL_CHUNKS)):
        copies[g] = issue(g)

    res = jnp.zeros((L,), jnp.float32)
    neg = jnp.full((L,), -jnp.inf, jnp.float32)
    m1l = m2l = None
    for g in range(TOTAL_CHUNKS):
        if g + NBUF - 1 < TOTAL_CHUNKS:
            copies[g + NBUF - 1] = issue(g + NBUF - 1)
        copies[g].wait()
        chunk_ref = bufs[g % NBUF]

        if g % CHUNKS_PER_ROW == 0:
            m1l = tuple([neg] * ACC)
            m2l = tuple([neg] * ACC)

        @plsc.parallel_loop(0, CSTEPS, unroll=2, carry=(m1l, m2l))
        def _chunk_loop(i, carry, chunk_ref=chunk_ref):
            m1s, m2s = carry
            n1, n2 = [], []
            for a in range(ACC):
                x = chunk_ref[pl.ds((i * ACC + a) * L, L)]
                n1.append(jnp.maximum(m1s[a], x))
                n2.append(jnp.maximum(m2s[a], jnp.minimum(m1s[a], x)))
            return tuple(n1), tuple(n2)

        m1l, m2l = _chunk_loop

        if g % CHUNKS_PER_ROW == CHUNKS_PER_ROW - 1:
            j = g // CHUNKS_PER_ROW
            p1, p2 = list(m1l), list(m2l)
            while len(p1) > 1:
                n1, n2 = [], []
                for a in range(0, len(p1), 2):
                    c1, c2 = _combine(p1[a], p2[a], p1[a + 1], p2[a + 1])
                    n1.append(c1)
                    n2.append(c2)
                p1, p2 = n1, n2
            m1, m2 = p1[0], p2[0]

            s1v = _bcast_max(m1)
            maskv = m1 == s1v
            cntv = plsc.all_reduce_population_count(maskv)
            t = jnp.where(maskv, m2, m1)
            s2v = jnp.where(cntv >= 2, s1v, _bcast_max(t))

            lane = lax.iota(jnp.int32, L)
            res = jnp.where(lane == j, s1v, res)
            res = jnp.where(lane == ROWS_PER_W + j, s2v, res)

    res_v[...] = res
    pltpu.sync_copy(res_v, out_hbm.at[wid])


TC_CHUNK = 1024


def _tc_top2_pair(x_ref, ncols):
    neg = jnp.full((TC_BLK, TC_CHUNK), -jnp.inf, jnp.float32)

    def step(c, carry):
        m1, m2 = carry
        x = x_ref[:, pl.ds(c * TC_CHUNK, TC_CHUNK)]
        n1 = jnp.maximum(m1, x)
        n2 = jnp.maximum(m2, jnp.minimum(m1, x))
        return n1, n2

    m1, m2 = lax.fori_loop(0, ncols // TC_CHUNK, step, (neg, neg))
    s1 = jnp.max(m1, axis=1, keepdims=True)
    eq = m1 == s1
    cnt = jnp.sum(eq.astype(jnp.float32), axis=1, keepdims=True)
    t = jnp.where(eq, m2, m1)
    s2raw = jnp.max(t, axis=1, keepdims=True)
    s2 = jnp.where(cnt >= 2.0, s1, s2raw)
    return s1, s2


def _top2_margin_tc_body(x_ref, x2_ref, o_ref, o2_ref):
    s1, s2 = _tc_top2_pair(x_ref, N)
    o_ref[...] = jnp.broadcast_to(s1 - s2, (TC_BLK, 128))

    p1, p2 = _tc_top2_pair(x2_ref, N - SC_COLS)
    ci = lax.broadcasted_iota(jnp.int32, (TC_BLK, 128), 1)
    o2_ref[...] = jnp.where(ci == 0, p1, jnp.where(ci == 1, p2, 0.0))


_tc_call = pl.pallas_call(
    _top2_margin_tc_body,
    grid=(TC_ROWS // TC_BLK,),
    in_specs=[
        pl.BlockSpec((TC_BLK, N), lambda i: (i + SC_ROWS // TC_BLK, 0)),
        pl.BlockSpec((TC_BLK, N - SC_COLS),
                     lambda i: (i, SC_COLS // (N - SC_COLS))),
    ],
    out_specs=[
        pl.BlockSpec((TC_BLK, 128), lambda i: (i, 0)),
        pl.BlockSpec((TC_BLK, 128), lambda i: (i, 0)),
    ],
    out_shape=[
        jax.ShapeDtypeStruct((TC_ROWS, 128), jnp.float32),
        jax.ShapeDtypeStruct((SC_ROWS, 128), jnp.float32),
    ],
)


def kernel(logits):
    sc = _top2_margin_sc(logits)
    tc, tc2 = _tc_call(logits, logits)
    a1 = sc[:, :ROWS_PER_W].reshape(SC_ROWS)
    a2 = sc[:, ROWS_PER_W:2 * ROWS_PER_W].reshape(SC_ROWS)
    b1 = tc2[:, 0]
    b2 = tc2[:, 1]
    hi = jnp.maximum(a1, b1)
    lo = jnp.maximum(jnp.minimum(a1, b1), jnp.maximum(a2, b2))
    return jnp.concatenate([hi - lo, tc[:, 0]])


# speedup vs baseline: 3.8937x; 1.0010x over previous
import functools

import jax
import jax.numpy as jnp
from jax import lax
from jax.experimental import pallas as pl
from jax.experimental.pallas import tpu as pltpu
from jax.experimental.pallas import tpu_sc as plsc

R = 128
N = 32768
L = 16
NW = 32
SC_ROWS = 64
TC_ROWS = R - SC_ROWS
ROWS_PER_W = SC_ROWS // NW
ACC = 8

TC_BLK = 16


def _bcast_max(x):
    fwd = plsc.cummax(x)
    bwd = lax.rev(plsc.cummax(lax.rev(x, (0,))), (0,))
    return jnp.maximum(fwd, bwd)


def _combine(a1, a2, b1, b2):
    n1 = jnp.maximum(a1, b1)
    n2 = jnp.maximum(jnp.minimum(a1, b1), jnp.maximum(a2, b2))
    return n1, n2


_mesh = plsc.VectorSubcoreMesh(core_axis_name="c", subcore_axis_name="s")


NBUF = 3
CHUNK = 8192
SC_COLS = 24576
CHUNKS_PER_ROW = SC_COLS // CHUNK
CSTEPS = CHUNK // (L * ACC)
TOTAL_CHUNKS = ROWS_PER_W * CHUNKS_PER_ROW


@functools.partial(
    pl.kernel,
    mesh=_mesh,
    out_type=jax.ShapeDtypeStruct((NW, L), jnp.float32),
    scratch_types=[
        pltpu.VMEM((CHUNK,), jnp.float32),
        pltpu.VMEM((CHUNK,), jnp.float32),
        pltpu.VMEM((CHUNK,), jnp.float32),
        pltpu.VMEM((L,), jnp.float32),
        pltpu.SemaphoreType.DMA,
        pltpu.SemaphoreType.DMA,
        pltpu.SemaphoreType.DMA,
    ],
    compiler_params=pltpu.CompilerParams(needs_layout_passes=False),
)
def _top2_margin_sc(logits_hbm, out_hbm, buf0, buf1, buf2, res_v,
                    sem0, sem1, sem2):
    cid = lax.axis_index("c")
    sid = lax.axis_index("s")
    wid = cid * 16 + sid
    base = wid * ROWS_PER_W
    sems = (sem0, sem1, sem2)
    bufs = (buf0, buf1, buf2)

    def issue(g):
        row_ref = logits_hbm.at[base + g // CHUNKS_PER_ROW]
        src = row_ref.at[pl.ds((g % CHUNKS_PER_ROW) * CHUNK, CHUNK)]
        return pltpu.async_copy(src, bufs[g % NBUF], sems[g % NBUF])

    copies = {}
    for g in range(min(NBUF - 1, TOTAL_CHUNKS)):
        copies[g] = issue(g)

    res = jnp.zeros((L,), jnp.float32)
    neg = jnp.full((L,), -jnp.inf, jnp.float32)
    m1l = m2l = None
    for g in range(TOTAL_CHUNKS):
        if g + NBUF - 1 < TOTAL_CHUNKS:
            copies[g + NBUF - 1] = issue(g + NBUF - 1)
        copies[g].wait()
        chunk_ref = bufs[g % NBUF]

        if g % CHUNKS_PER_ROW == 0:
            m1l = tuple([neg] * ACC)
            m2l = tuple([neg] * ACC)

        @plsc.parallel_loop(0, CSTEPS, unroll=2, carry=(m1l, m2l))
        def _chunk_loop(i, carry, chunk_ref=chunk_ref):
            m1s, m2s = carry
            n1, n2 = [], []
            for a in range(ACC):
                x = chunk_ref[pl.ds((i * ACC + a) * L, L)]
                n1.append(jnp.maximum(m1s[a], x))
                n2.append(jnp.maximum(m2s[a], jnp.minimum(m1s[a], x)))
            return tuple(n1), tuple(n2)

        m1l, m2l = _chunk_loop

        if g % CHUNKS_PER_ROW == CHUNKS_PER_ROW - 1:
            j = g // CHUNKS_PER_ROW
            p1, p2 = list(m1l), list(m2l)
            while len(p1) > 1:
                n1, n2 = [], []
                for a in range(0, len(p1), 2):
                    c1, c2 = _combine(p1[a], p2[a], p1[a + 1], p2[a + 1])
                    n1.append(c1)
                    n2.append(c2)
                p1, p2 = n1, n2
            m1, m2 = p1[0], p2[0]

            s1v = _bcast_max(m1)
            maskv = m1 == s1v
            cntv = plsc.all_reduce_population_count(maskv)
            t = jnp.where(maskv, m2, m1)
            s2v = jnp.where(cntv >= 2, s1v, _bcast_max(t))

            lane = lax.iota(jnp.int32, L)
            res = jnp.where(lane == j, s1v, res)
            res = jnp.where(lane == ROWS_PER_W + j, s2v, res)

    res_v[...] = res
    pltpu.sync_copy(res_v, out_hbm.at[wid])


TC_CHUNK = 1024


def _tc_top2_pair(x_ref, ncols):
    neg = jnp.full((TC_BLK, TC_CHUNK), -jnp.inf, jnp.float32)

    def step(c, carry):
        m1, m2 = carry
        x = x_ref[:, pl.ds(c * TC_CHUNK, TC_CHUNK)]
        n1 = jnp.maximum(m1, x)
        n2 = jnp.maximum(m2, jnp.minimum(m1, x))
        return n1, n2

    m1, m2 = lax.fori_loop(0, ncols // TC_CHUNK, step, (neg, neg))
    s1 = jnp.max(m1, axis=1, keepdims=True)
    eq = m1 == s1
    cnt = jnp.sum(eq.astype(jnp.float32), axis=1, keepdims=True)
    t = jnp.where(eq, m2, m1)
    s2raw = jnp.max(t, axis=1, keepdims=True)
    s2 = jnp.where(cnt >= 2.0, s1, s2raw)
    return s1, s2


def _top2_margin_tc_body(x_ref, x2_ref, o_ref, o2_ref):
    s1, s2 = _tc_top2_pair(x_ref, N)
    o_ref[...] = jnp.broadcast_to(s1 - s2, (TC_BLK, 128))

    p1, p2 = _tc_top2_pair(x2_ref, N - SC_COLS)
    ci = lax.broadcasted_iota(jnp.int32, (TC_BLK, 128), 1)
    o2_ref[...] = jnp.where(ci == 0, p1, jnp.where(ci == 1, p2, 0.0))


_tc_call = pl.pallas_call(
    _top2_margin_tc_body,
    grid=(TC_ROWS // TC_BLK,),
    in_specs=[
        pl.BlockSpec((TC_BLK, N), lambda i: (i + SC_ROWS // TC_BLK, 0)),
        pl.BlockSpec((TC_BLK, N - SC_COLS),
                     lambda i: (i, SC_COLS // (N - SC_COLS))),
    ],
    out_specs=[
        pl.BlockSpec((TC_BLK, 128), lambda i: (i, 0)),
        pl.BlockSpec((TC_BLK, 128), lambda i: (i, 0)),
    ],
    out_shape=[
        jax.ShapeDtypeStruct((TC_ROWS, 128), jnp.float32),
        jax.ShapeDtypeStruct((SC_ROWS, 128), jnp.float32),
    ],
)


def kernel(logits):
    sc = _top2_margin_sc(logits)
    tc, tc2 = _tc_call(logits, logits)
    a1 = sc[:, :ROWS_PER_W].reshape(SC_ROWS)
    a2 = sc[:, ROWS_PER_W:2 * ROWS_PER_W].reshape(SC_ROWS)
    b1 = tc2[:, 0]
    b2 = tc2[:, 1]
    hi = jnp.maximum(a1, b1)
    lo = jnp.maximum(jnp.minimum(a1, b1), jnp.maximum(a2, b2))
    return jnp.concatenate([hi - lo, tc[:, 0]])
